# baseline (speedup 1.0000x reference)
import functools

import jax
import jax.numpy as jnp
from jax import lax
from jax.experimental import pallas as pl
from jax.experimental.pallas import tpu as pltpu
from jax.experimental.pallas import tpu_sc as plsc

H = 1024
E = 64
I = 256
G = 8
TKG = 4
K = 2
IS = 512
T = 2048
EPG = E // G
TM = 128
NT = (T * K) // TM + E
NS = NT * TM
NA = T * K
NW = 32
RPW = NS // NW
TPW = T // NW


def _gate_kernel(x_ref, wgate_ref, bias_ref, wsg_ref, wsu_ref, wsd_ref,
                 slot0_ref, slot1_ref, w0_ref, w1_ref, te_ref, nv_ref,
                 shared_ref):
    x = x_ref[...]
    logits = jnp.dot(x, wgate_ref[...], preferred_element_type=jnp.float32)
    scores = jax.nn.sigmoid(logits) + bias_ref[...]
    gs = jnp.concatenate(
        [jnp.max(scores[:, g * EPG:(g + 1) * EPG], axis=1, keepdims=True)
         for g in range(G)], axis=1)
    giota = lax.broadcasted_iota(jnp.int32, (T, G), 1)
    gmask = jnp.zeros((T, G), jnp.float32)
    cur = gs
    for _ in range(TKG):
        m = jnp.max(cur, axis=1, keepdims=True)
        sel_idx = jnp.min(jnp.where(cur == m, giota, G), axis=1, keepdims=True)
        sel = giota == sel_idx
        gmask = gmask + sel.astype(jnp.float32)
        cur = jnp.where(sel, -jnp.inf, cur)
    emask = jnp.concatenate(
        [jnp.broadcast_to(gmask[:, g:g + 1], (T, EPG)) for g in range(G)],
        axis=1)
    masked = scores * emask
    eiota = lax.broadcasted_iota(jnp.int32, (T, E), 1)
    cur = masked
    ws, sels = [], []
    for _ in range(K):
        m = jnp.max(cur, axis=1, keepdims=True)
        si = jnp.min(jnp.where(cur == m, eiota, E), axis=1, keepdims=True)
        sel = (eiota == si).astype(jnp.float32)
        ws.append(m)
        sels.append(sel)
        cur = jnp.where(sel > 0, -jnp.inf, cur)
    denom = ws[0] + ws[1] + 1e-8

    cnt = sels[0] + sels[1]
    inc = cnt
    d = 1
    while d < T:
        inc = inc + jnp.concatenate(
            [jnp.zeros((d, E), jnp.float32), inc[:-d, :]], axis=0)
        d *= 2
    exc = jnp.concatenate([jnp.zeros((1, E), jnp.float32), inc[:-1, :]], axis=0)
    counts = inc[T - 1:T, :].astype(jnp.int32)
    tiles = jnp.right_shift(counts + (TM - 1), 7)
    acc = tiles
    d = 1
    while d < E:
        acc = acc + jnp.concatenate(
            [jnp.zeros((1, d), jnp.int32), acc[:, :-d]], axis=1)
        d *= 2
    tstart = acc - tiles
    nv = jnp.sum(tiles, axis=1, keepdims=True)
    po = (tstart * TM).astype(jnp.float32)

    slots = []
    for k in range(K):
        rank = jnp.sum(exc * sels[k], axis=1, keepdims=True)
        base = jnp.sum(po * sels[k], axis=1, keepdims=True)
        slots.append((base + rank).astype(jnp.int32))
    slot0_ref[...] = slots[0]
    slot1_ref[...] = slots[1]
    w0_ref[...] = ws[0] / denom
    w1_ref[...] = ws[1] / denom

    i_row = lax.broadcasted_iota(jnp.int32, (1, 128), 1)
    i_row = jnp.minimum(i_row, nv - 1)
    ident = (lax.broadcasted_iota(jnp.int32, (E, E), 0)
             == lax.broadcasted_iota(jnp.int32, (E, E), 1)).astype(jnp.int32)
    tstart_col = jnp.sum(tstart * ident, axis=1, keepdims=True)
    te_ref[...] = jnp.sum((tstart_col <= i_row).astype(jnp.int32),
                          axis=0, keepdims=True) - 1
    nv_ref[...] = nv

    x16 = x.astype(jnp.bfloat16)
    g = jnp.dot(x16, wsg_ref[...].astype(jnp.bfloat16),
                preferred_element_type=jnp.float32)
    u = jnp.dot(x16, wsu_ref[...].astype(jnp.bfloat16),
                preferred_element_type=jnp.float32)
    h = (jax.nn.silu(g) * u).astype(jnp.bfloat16)
    shared_ref[...] = jnp.dot(h, wsd_ref[...].astype(jnp.bfloat16),
                              preferred_element_type=jnp.float32)


def _dispatch_sc_kernel(x_hbm, s0_hbm, s1_hbm, w0_hbm, w1_hbm, xs_hbm,
                        wsort_hbm, s0_v, s1_v, w0_v, w1_v, wsort_v, idx2_v,
                        rows_v, sem):
    c = lax.axis_index("c")
    s = lax.axis_index("s")

    @pl.when(jnp.logical_and(s == 0, c == 0))
    def _wsort():
        pltpu.sync_copy(s0_hbm, s0_v)
        pltpu.sync_copy(s1_hbm, s1_v)
        pltpu.sync_copy(w0_hbm, w0_v)
        pltpu.sync_copy(w1_hbm, w1_v)
        zero_f = jnp.zeros((16,), jnp.float32)

        def _zero(j, carry):
            wsort_v[pl.ds(j * 16, 16)] = zero_f
            return carry

        lax.fori_loop(0, NS // 16, _zero, 0)

        def _scatter(j, carry):
            base = j * 16
            plsc.store_scatter(wsort_v, [s0_v[pl.ds(base, 16)]],
                               w0_v[pl.ds(base, 16)])
            plsc.store_scatter(wsort_v, [s1_v[pl.ds(base, 16)]],
                               w1_v[pl.ds(base, 16)])
            return carry

        lax.fori_loop(0, T // 16, _scatter, 0)
        pltpu.sync_copy(wsort_v, wsort_hbm)

    wid = c * 16 + s
    t0 = wid * TPW
    pltpu.sync_copy(s0_hbm.at[pl.ds(t0, TPW)], idx2_v.at[0])
    pltpu.sync_copy(s1_hbm.at[pl.ds(t0, TPW)], idx2_v.at[1])
    pltpu.sync_copy(x_hbm.at[pl.ds(t0, TPW)], rows_v)
    cp0 = pltpu.async_copy(rows_v, xs_hbm.at[idx2_v.at[0]], sem)
    cp1 = pltpu.async_copy(rows_v, xs_hbm.at[idx2_v.at[1]], sem)
    cp0.wait()
    cp1.wait()


def _gemm_kernel(te_ref, nv_ref, xs_ref, wg_ref, wu_ref, wd_ref, ws_ref,
                 y_ref):
    i = pl.program_id(0)

    @pl.when(i < nv_ref[0])
    def _():
        x = xs_ref[...].astype(jnp.bfloat16)
        g = jnp.dot(x, wg_ref[0].astype(jnp.bfloat16),
                    preferred_element_type=jnp.float32)
        u = jnp.dot(x, wu_ref[0].astype(jnp.bfloat16),
                    preferred_element_type=jnp.float32)
        h = (jax.nn.silu(g) * u).astype(jnp.bfloat16)
        y = jnp.dot(h, wd_ref[0].astype(jnp.bfloat16),
                    preferred_element_type=jnp.float32)
        ident = (lax.broadcasted_iota(jnp.int32, (TM, TM), 0)
                 == lax.broadcasted_iota(jnp.int32, (TM, TM), 1)
                 ).astype(jnp.float32)
        wcol = jnp.sum(ws_ref[0] * ident, axis=1, keepdims=True)
        y_ref[...] = y * wcol


def _combine_sc_kernel(s0_hbm, s1_hbm, y_hbm, shared_hbm, out_hbm,
                       idx2_v, rows_v, sh_v, sem):
    c = lax.axis_index("c")
    s = lax.axis_index("s")
    wid = c * 16 + s
    tok0 = wid * TPW
    HT = TPW // 2
    pltpu.sync_copy(s0_hbm.at[pl.ds(tok0, TPW)], idx2_v.at[0])
    pltpu.sync_copy(s1_hbm.at[pl.ds(tok0, TPW)], idx2_v.at[1])
    for half in range(2):
        t0 = tok0 + half * HT
        toff = half * HT
        g0 = pltpu.async_copy(
            y_hbm.at[idx2_v.at[0, pl.ds(toff, HT)]],
            rows_v.at[pl.ds(0, HT)], sem)
        g1 = pltpu.async_copy(
            y_hbm.at[idx2_v.at[1, pl.ds(toff, HT)]],
            rows_v.at[pl.ds(HT, HT)], sem)
        pltpu.sync_copy(shared_hbm.at[pl.ds(t0, HT)], sh_v)
        g0.wait()
        g1.wait()

        def _tok(t, carry):
            def _chunk(j, carry2):
                cs = pl.ds(j * 16, 16)
                sh_v[t, cs] = (sh_v[t, cs] + rows_v[t, cs]
                               + rows_v[HT + t, cs])
                return carry2

            lax.fori_loop(0, H // 16, _chunk, 0)
            return carry

        lax.fori_loop(0, HT, _tok, 0)
        pltpu.sync_copy(sh_v, out_hbm.at[pl.ds(t0, HT)])


def kernel(hidden_states, W_gate, bias_corr, Wg, Wu, Wd, Ws_g, Ws_u, Ws_d):
    x = hidden_states.reshape(T, H)
    bias2d = bias_corr.reshape(1, E)

    slot0, slot1, w0, w1, te, nv, shared = pl.pallas_call(
        _gate_kernel,
        out_shape=(
            jax.ShapeDtypeStruct((T, 1), jnp.int32),
            jax.ShapeDtypeStruct((T, 1), jnp.int32),
            jax.ShapeDtypeStruct((T, 1), jnp.float32),
            jax.ShapeDtypeStruct((T, 1), jnp.float32),
            jax.ShapeDtypeStruct((1, 128), jnp.int32),
            jax.ShapeDtypeStruct((1, 1), jnp.int32),
            jax.ShapeDtypeStruct((T, H), jnp.float32),
        ),
    )(x, W_gate, bias2d, Ws_g, Ws_u, Ws_d)

    s0 = slot0.reshape(T)
    s1 = slot1.reshape(T)
    w0f = w0.reshape(T)
    w1f = w1.reshape(T)

    mesh = plsc.VectorSubcoreMesh(core_axis_name="c", subcore_axis_name="s",
                                  num_cores=2, num_subcores=16)
    sc_params = pltpu.CompilerParams(needs_layout_passes=False)
    dispatch = functools.partial(
        pl.kernel, _dispatch_sc_kernel, mesh=mesh,
        compiler_params=sc_params,
        out_type=(
            jax.ShapeDtypeStruct((NS, H), jnp.float32),
            jax.ShapeDtypeStruct((NS,), jnp.float32),
        ),
        scratch_types=[
            pltpu.VMEM((T,), jnp.int32),
            pltpu.VMEM((T,), jnp.int32),
            pltpu.VMEM((T,), jnp.float32),
            pltpu.VMEM((T,), jnp.float32),
            pltpu.VMEM((NS,), jnp.float32),
            pltpu.VMEM((K, TPW), jnp.int32),
            pltpu.VMEM((TPW, H), jnp.float32),
            pltpu.SemaphoreType.DMA,
        ],
    )()
    xs, wsort = dispatch(x, s0, s1, w0f, w1f)

    y = pl.pallas_call(
        _gemm_kernel,
        grid_spec=pltpu.PrefetchScalarGridSpec(
            num_scalar_prefetch=2,
            grid=(NT,),
            in_specs=[
                pl.BlockSpec((TM, H),
                             lambda i, te, nv: (jnp.minimum(i, nv[0] - 1), 0)),
                pl.BlockSpec((1, H, I), lambda i, te, nv: (te[i], 0, 0)),
                pl.BlockSpec((1, H, I), lambda i, te, nv: (te[i], 0, 0)),
                pl.BlockSpec((1, I, H), lambda i, te, nv: (te[i], 0, 0)),
                pl.BlockSpec((1, 1, TM),
                             lambda i, te, nv: (jnp.minimum(i, nv[0] - 1), 0, 0)),
            ],
            out_specs=pl.BlockSpec(
                (TM, H),
                lambda i, te, nv: (jnp.where(i < nv[0], i, NT - 1), 0)),
        ),
        out_shape=jax.ShapeDtypeStruct((NS, H), jnp.float32),
    )(te.reshape(128), nv.reshape(1), xs, Wg, Wu, Wd,
      wsort.reshape(NT, 1, TM))

    combine = functools.partial(
        pl.kernel, _combine_sc_kernel, mesh=mesh,
        compiler_params=sc_params,
        out_type=jax.ShapeDtypeStruct((T, H), jnp.float32),
        scratch_types=[
            pltpu.VMEM((K, TPW), jnp.int32),
            pltpu.VMEM((TPW, H), jnp.float32),
            pltpu.VMEM((TPW // 2, H), jnp.float32),
            pltpu.SemaphoreType.DMA,
        ],
    )()
    out = combine(s0, s1, y, shared)

    return out.reshape(1, T, H)


# speedup vs baseline: 1.0038x; 1.0038x over previous
import functools

import jax
import jax.numpy as jnp
from jax import lax
from jax.experimental import pallas as pl
from jax.experimental.pallas import tpu as pltpu
from jax.experimental.pallas import tpu_sc as plsc

H = 1024
E = 64
I = 256
G = 8
TKG = 4
K = 2
IS = 512
T = 2048
EPG = E // G
TM = 128
NT = (T * K) // TM + E
NS = NT * TM
NA = T * K
NW = 32
RPW = NS // NW
TPW = T // NW


def _gate_kernel(x_ref, wgate_ref, bias_ref,
                 slot0_ref, slot1_ref, w0_ref, w1_ref, te_ref, nv_ref):
    x = x_ref[...]
    logits = jnp.dot(x, wgate_ref[...], preferred_element_type=jnp.float32)
    scores = jax.nn.sigmoid(logits) + bias_ref[...]
    gs = jnp.concatenate(
        [jnp.max(scores[:, g * EPG:(g + 1) * EPG], axis=1, keepdims=True)
         for g in range(G)], axis=1)
    giota = lax.broadcasted_iota(jnp.int32, (T, G), 1)
    gmask = jnp.zeros((T, G), jnp.float32)
    cur = gs
    for _ in range(TKG):
        m = jnp.max(cur, axis=1, keepdims=True)
        sel_idx = jnp.min(jnp.where(cur == m, giota, G), axis=1, keepdims=True)
        sel = giota == sel_idx
        gmask = gmask + sel.astype(jnp.float32)
        cur = jnp.where(sel, -jnp.inf, cur)
    emask = jnp.concatenate(
        [jnp.broadcast_to(gmask[:, g:g + 1], (T, EPG)) for g in range(G)],
        axis=1)
    masked = scores * emask
    eiota = lax.broadcasted_iota(jnp.int32, (T, E), 1)
    cur = masked
    ws, sels = [], []
    for _ in range(K):
        m = jnp.max(cur, axis=1, keepdims=True)
        si = jnp.min(jnp.where(cur == m, eiota, E), axis=1, keepdims=True)
        sel = (eiota == si).astype(jnp.float32)
        ws.append(m)
        sels.append(sel)
        cur = jnp.where(sel > 0, -jnp.inf, cur)
    denom = ws[0] + ws[1] + 1e-8

    cnt = sels[0] + sels[1]
    inc = cnt
    d = 1
    while d < T:
        inc = inc + jnp.concatenate(
            [jnp.zeros((d, E), jnp.float32), inc[:-d, :]], axis=0)
        d *= 2
    exc = jnp.concatenate([jnp.zeros((1, E), jnp.float32), inc[:-1, :]], axis=0)
    counts = inc[T - 1:T, :].astype(jnp.int32)
    tiles = jnp.right_shift(counts + (TM - 1), 7)
    acc = tiles
    d = 1
    while d < E:
        acc = acc + jnp.concatenate(
            [jnp.zeros((1, d), jnp.int32), acc[:, :-d]], axis=1)
        d *= 2
    tstart = acc - tiles
    nv = jnp.sum(tiles, axis=1, keepdims=True)
    po = (tstart * TM).astype(jnp.float32)

    slots = []
    for k in range(K):
        rank = jnp.sum(exc * sels[k], axis=1, keepdims=True)
        base = jnp.sum(po * sels[k], axis=1, keepdims=True)
        slots.append((base + rank).astype(jnp.int32))
    slot0_ref[...] = slots[0]
    slot1_ref[...] = slots[1]
    w0_ref[...] = ws[0] / denom
    w1_ref[...] = ws[1] / denom

    i_row = lax.broadcasted_iota(jnp.int32, (1, 128), 1)
    i_row = jnp.minimum(i_row, nv - 1)
    ident = (lax.broadcasted_iota(jnp.int32, (E, E), 0)
             == lax.broadcasted_iota(jnp.int32, (E, E), 1)).astype(jnp.int32)
    tstart_col = jnp.sum(tstart * ident, axis=1, keepdims=True)
    te_ref[...] = jnp.sum((tstart_col <= i_row).astype(jnp.int32),
                          axis=0, keepdims=True) - 1
    nv_ref[...] = nv


def _shared_kernel(x_ref, wsg_ref, wsu_ref, wsd_ref, shared_ref):
    x16 = x_ref[...].astype(jnp.bfloat16)
    g = jnp.dot(x16, wsg_ref[...].astype(jnp.bfloat16),
                preferred_element_type=jnp.float32)
    u = jnp.dot(x16, wsu_ref[...].astype(jnp.bfloat16),
                preferred_element_type=jnp.float32)
    h = (jax.nn.silu(g) * u).astype(jnp.bfloat16)
    shared_ref[...] = jnp.dot(h, wsd_ref[...].astype(jnp.bfloat16),
                              preferred_element_type=jnp.float32)


def _dispatch_sc_kernel(x_hbm, s0_hbm, s1_hbm, w0_hbm, w1_hbm, xs_hbm,
                        wsort_hbm, s0_v, s1_v, w0_v, w1_v, wsort_v, idx2_v,
                        rows_v, sem):
    c = lax.axis_index("c")
    s = lax.axis_index("s")

    @pl.when(jnp.logical_and(s == 0, c == 0))
    def _wsort():
        pltpu.sync_copy(s0_hbm, s0_v)
        pltpu.sync_copy(s1_hbm, s1_v)
        pltpu.sync_copy(w0_hbm, w0_v)
        pltpu.sync_copy(w1_hbm, w1_v)
        zero_f = jnp.zeros((16,), jnp.float32)

        def _zero(j, carry):
            wsort_v[pl.ds(j * 16, 16)] = zero_f
            return carry

        lax.fori_loop(0, NS // 16, _zero, 0, unroll=8)

        def _scatter(j, carry):
            base = j * 16
            plsc.store_scatter(wsort_v, [s0_v[pl.ds(base, 16)]],
                               w0_v[pl.ds(base, 16)])
            plsc.store_scatter(wsort_v, [s1_v[pl.ds(base, 16)]],
                               w1_v[pl.ds(base, 16)])
            return carry

        lax.fori_loop(0, T // 16, _scatter, 0, unroll=4)
        pltpu.sync_copy(wsort_v, wsort_hbm)

    wid = c * 16 + s
    t0 = wid * TPW
    pltpu.sync_copy(s0_hbm.at[pl.ds(t0, TPW)], idx2_v.at[0])
    pltpu.sync_copy(s1_hbm.at[pl.ds(t0, TPW)], idx2_v.at[1])
    pltpu.sync_copy(x_hbm.at[pl.ds(t0, TPW)], rows_v)
    cp0 = pltpu.async_copy(rows_v, xs_hbm.at[idx2_v.at[0]], sem)
    cp1 = pltpu.async_copy(rows_v, xs_hbm.at[idx2_v.at[1]], sem)
    cp0.wait()
    cp1.wait()


def _gemm_kernel(te_ref, nv_ref, xs_ref, wg_ref, wu_ref, wd_ref, ws_ref,
                 y_ref):
    i = pl.program_id(0)

    @pl.when(i < nv_ref[0])
    def _():
        x = xs_ref[...].astype(jnp.bfloat16)
        g = jnp.dot(x, wg_ref[0].astype(jnp.bfloat16),
                    preferred_element_type=jnp.float32)
        u = jnp.dot(x, wu_ref[0].astype(jnp.bfloat16),
                    preferred_element_type=jnp.float32)
        h = (jax.nn.silu(g) * u).astype(jnp.bfloat16)
        y = jnp.dot(h, wd_ref[0].astype(jnp.bfloat16),
                    preferred_element_type=jnp.float32)
        ident = (lax.broadcasted_iota(jnp.int32, (TM, TM), 0)
                 == lax.broadcasted_iota(jnp.int32, (TM, TM), 1)
                 ).astype(jnp.float32)
        wcol = jnp.sum(ws_ref[0] * ident, axis=1, keepdims=True)
        y_ref[...] = y * wcol


def _combine_sc_kernel(s0_hbm, s1_hbm, y_hbm, shared_hbm, out_hbm,
                       idx2_v, rows_v, sh_v, sem):
    c = lax.axis_index("c")
    s = lax.axis_index("s")
    wid = c * 16 + s
    tok0 = wid * TPW
    HT = TPW // 2
    pltpu.sync_copy(s0_hbm.at[pl.ds(tok0, TPW)], idx2_v.at[0])
    pltpu.sync_copy(s1_hbm.at[pl.ds(tok0, TPW)], idx2_v.at[1])
    for half in range(2):
        t0 = tok0 + half * HT
        toff = half * HT
        g0 = pltpu.async_copy(
            y_hbm.at[idx2_v.at[0, pl.ds(toff, HT)]],
            rows_v.at[pl.ds(0, HT)], sem)
        g1 = pltpu.async_copy(
            y_hbm.at[idx2_v.at[1, pl.ds(toff, HT)]],
            rows_v.at[pl.ds(HT, HT)], sem)
        pltpu.sync_copy(shared_hbm.at[pl.ds(t0, HT)], sh_v)
        g0.wait()
        g1.wait()

        def _tok(t, carry):
            def _chunk(j, carry2):
                cs = pl.ds(j * 16, 16)
                sh_v[t, cs] = (sh_v[t, cs] + rows_v[t, cs]
                               + rows_v[HT + t, cs])
                return carry2

            lax.fori_loop(0, H // 16, _chunk, 0, unroll=8)
            return carry

        lax.fori_loop(0, HT, _tok, 0)
        pltpu.sync_copy(sh_v, out_hbm.at[pl.ds(t0, HT)])


def kernel(hidden_states, W_gate, bias_corr, Wg, Wu, Wd, Ws_g, Ws_u, Ws_d):
    x = hidden_states.reshape(T, H)
    bias2d = bias_corr.reshape(1, E)

    slot0, slot1, w0, w1, te, nv = pl.pallas_call(
        _gate_kernel,
        out_shape=(
            jax.ShapeDtypeStruct((T, 1), jnp.int32),
            jax.ShapeDtypeStruct((T, 1), jnp.int32),
            jax.ShapeDtypeStruct((T, 1), jnp.float32),
            jax.ShapeDtypeStruct((T, 1), jnp.float32),
            jax.ShapeDtypeStruct((1, 128), jnp.int32),
            jax.ShapeDtypeStruct((1, 1), jnp.int32),
        ),
    )(x, W_gate, bias2d)

    s0 = slot0.reshape(T)
    s1 = slot1.reshape(T)
    w0f = w0.reshape(T)
    w1f = w1.reshape(T)

    mesh = plsc.VectorSubcoreMesh(core_axis_name="c", subcore_axis_name="s",
                                  num_cores=2, num_subcores=16)
    sc_params = pltpu.CompilerParams(needs_layout_passes=False)
    dispatch = functools.partial(
        pl.kernel, _dispatch_sc_kernel, mesh=mesh,
        compiler_params=sc_params,
        out_type=(
            jax.ShapeDtypeStruct((NS, H), jnp.float32),
            jax.ShapeDtypeStruct((NS,), jnp.float32),
        ),
        scratch_types=[
            pltpu.VMEM((T,), jnp.int32),
            pltpu.VMEM((T,), jnp.int32),
            pltpu.VMEM((T,), jnp.float32),
            pltpu.VMEM((T,), jnp.float32),
            pltpu.VMEM((NS,), jnp.float32),
            pltpu.VMEM((K, TPW), jnp.int32),
            pltpu.VMEM((TPW, H), jnp.float32),
            pltpu.SemaphoreType.DMA,
        ],
    )()
    xs, wsort = dispatch(x, s0, s1, w0f, w1f)

    shared = pl.pallas_call(
        _shared_kernel,
        out_shape=jax.ShapeDtypeStruct((T, H), jnp.float32),
    )(x, Ws_g, Ws_u, Ws_d)

    y = pl.pallas_call(
        _gemm_kernel,
        grid_spec=pltpu.PrefetchScalarGridSpec(
            num_scalar_prefetch=2,
            grid=(NT,),
            in_specs=[
                pl.BlockSpec((TM, H),
                             lambda i, te, nv: (jnp.minimum(i, nv[0] - 1), 0)),
                pl.BlockSpec((1, H, I), lambda i, te, nv: (te[i], 0, 0)),
                pl.BlockSpec((1, H, I), lambda i, te, nv: (te[i], 0, 0)),
                pl.BlockSpec((1, I, H), lambda i, te, nv: (te[i], 0, 0)),
                pl.BlockSpec((1, 1, TM),
                             lambda i, te, nv: (jnp.minimum(i, nv[0] - 1), 0, 0)),
            ],
            out_specs=pl.BlockSpec(
                (TM, H),
                lambda i, te, nv: (jnp.where(i < nv[0], i, NT - 1), 0)),
        ),
        out_shape=jax.ShapeDtypeStruct((NS, H), jnp.float32),
    )(te.reshape(128), nv.reshape(1), xs, Wg, Wu, Wd,
      wsort.reshape(NT, 1, TM))

    combine = functools.partial(
        pl.kernel, _combine_sc_kernel, mesh=mesh,
        compiler_params=sc_params,
        out_type=jax.ShapeDtypeStruct((T, H), jnp.float32),
        scratch_types=[
            pltpu.VMEM((K, TPW), jnp.int32),
            pltpu.VMEM((TPW, H), jnp.float32),
            pltpu.VMEM((TPW // 2, H), jnp.float32),
            pltpu.SemaphoreType.DMA,
        ],
    )()
    out = combine(s0, s1, y, shared)

    return out.reshape(1, T, H)
